# baseline (speedup 1.0000x reference)
import functools

import jax
import jax.numpy as jnp
from jax import lax
from jax.experimental import pallas as pl
from jax.experimental.pallas import tpu as pltpu
from jax.experimental.pallas import tpu_sc as plsc

B, D = 4096, 12288

_NC, _NS = 2, 16
_NW = _NC * _NS
_CHUNK = B // _NW
_LANES = 16


def _make_sc_gather(r0, nrows):
    chunk = nrows // _NW
    mesh = plsc.VectorSubcoreMesh(core_axis_name="c", subcore_axis_name="s")

    @functools.partial(
        pl.kernel,
        mesh=mesh,
        out_type=(
            jax.ShapeDtypeStruct((nrows,), jnp.float32),
            jax.ShapeDtypeStruct((nrows,), jnp.float32),
        ),
        scratch_types=[
            pltpu.VMEM((chunk,), jnp.int32),
            pltpu.VMEM((chunk,), jnp.float32),
            pltpu.VMEM((chunk,), jnp.float32),
            pltpu.SemaphoreType.DMA,
            pltpu.SemaphoreType.DMA,
        ],
    )
    def sc_gather(t_hbm, sa_hbm, sb_hbm, oa_hbm, ob_hbm,
                  idx_v, oa_v, ob_v, sem_a, sem_b):
        wid = lax.axis_index("s") * _NC + lax.axis_index("c")
        base = wid * chunk
        pltpu.sync_copy(t_hbm.at[pl.ds(r0 + base, chunk)], idx_v)
        cp_a = pltpu.async_copy(sa_hbm.at[idx_v], oa_v, sem_a)
        cp_b = pltpu.async_copy(sb_hbm.at[idx_v], ob_v, sem_b)
        cp_a.wait()
        cp_b.wait()
        pltpu.sync_copy(oa_v, oa_hbm.at[pl.ds(base, chunk)])
        pltpu.sync_copy(ob_v, ob_hbm.at[pl.ds(base, chunk)])

    return sc_gather


_BR = 256
_BC = 4096


def _tc_body(sa_ref, sb_ref, x_ref, n_ref, o_ref, nc_ref):
    nv = n_ref[...]
    sa = sa_ref[...].reshape(_BR, 1)
    sb = sb_ref[...].reshape(_BR, 1)
    o_ref[...] = jnp.clip(sa * x_ref[...] + sb * nv, -1.0, 1.0)
    nc_ref[...] = nv


def _tc_tail_body(o_in, nc_in, sa_ref, sb_ref, x_ref, n_ref, o_ref, nc_ref):
    del o_in, nc_in
    _tc_body(sa_ref, sb_ref, x_ref, n_ref, o_ref, nc_ref)


R_HEAD = 1024


def kernel(x_0, t, noise, sqrt_alpha, sqrt_1m_alpha):
    sa_h, sb_h = _make_sc_gather(0, R_HEAD)(t, sqrt_alpha, sqrt_1m_alpha)
    sa_t, sb_t = _make_sc_gather(R_HEAD, B - R_HEAD)(
        t, sqrt_alpha, sqrt_1m_alpha)
    big = [
        jax.ShapeDtypeStruct((B, D), jnp.float32),
        jax.ShapeDtypeStruct((B, D), jnp.float32),
    ]
    blk = pl.BlockSpec((_BR, _BC), lambda i, j: (i, j))
    svec = pl.BlockSpec((_BR,), lambda i, j: (i,))
    o_head, nc_head = pl.pallas_call(
        _tc_body,
        grid=(R_HEAD // _BR, D // _BC),
        in_specs=[svec, svec, blk, blk],
        out_specs=[blk, blk],
        out_shape=big,
    )(sa_h, sb_h, x_0, noise)
    nh = R_HEAD // _BR
    blk_t = pl.BlockSpec((_BR, _BC), lambda i, j: (nh + i, j))
    svec_t = pl.BlockSpec((_BR,), lambda i, j: (i,))
    out, n_copy = pl.pallas_call(
        _tc_tail_body,
        grid=((B - R_HEAD) // _BR, D // _BC),
        in_specs=[
            pl.BlockSpec(memory_space=pl.ANY),
            pl.BlockSpec(memory_space=pl.ANY),
            svec_t, svec_t, blk_t, blk_t,
        ],
        out_specs=[blk_t, blk_t],
        out_shape=big,
        input_output_aliases={0: 0, 1: 1},
    )(o_head, nc_head, sa_t, sb_t, x_0, noise)
    return out, n_copy


# speedup vs baseline: 1.0038x; 1.0038x over previous
import functools

import jax
import jax.numpy as jnp
from jax import lax
from jax.experimental import pallas as pl
from jax.experimental.pallas import tpu as pltpu
from jax.experimental.pallas import tpu_sc as plsc

B, D = 4096, 12288

_NC, _NS = 2, 16
_NW = _NC * _NS
_CHUNK = B // _NW
_LANES = 16


def _make_sc_gather():
    mesh = plsc.VectorSubcoreMesh(core_axis_name="c", subcore_axis_name="s")

    @functools.partial(
        pl.kernel,
        mesh=mesh,
        out_type=(
            jax.ShapeDtypeStruct((B,), jnp.float32),
            jax.ShapeDtypeStruct((B,), jnp.float32),
        ),
        scratch_types=[
            pltpu.VMEM((_CHUNK,), jnp.int32),
            pltpu.VMEM((_CHUNK,), jnp.float32),
            pltpu.VMEM((_CHUNK,), jnp.float32),
            pltpu.SemaphoreType.DMA,
            pltpu.SemaphoreType.DMA,
        ],
    )
    def sc_gather(t_hbm, sa_hbm, sb_hbm, oa_hbm, ob_hbm,
                  idx_v, oa_v, ob_v, sem_a, sem_b):
        wid = lax.axis_index("s") * _NC + lax.axis_index("c")
        base = wid * _CHUNK
        pltpu.sync_copy(t_hbm.at[pl.ds(base, _CHUNK)], idx_v)
        cp_a = pltpu.async_copy(sa_hbm.at[idx_v], oa_v, sem_a)
        cp_b = pltpu.async_copy(sb_hbm.at[idx_v], ob_v, sem_b)
        cp_a.wait()
        cp_b.wait()
        pltpu.sync_copy(oa_v, oa_hbm.at[pl.ds(base, _CHUNK)])
        pltpu.sync_copy(ob_v, ob_hbm.at[pl.ds(base, _CHUNK)])

    return sc_gather


_BR = 256
_BC = 4096


def _tc_body(sa_ref, sb_ref, x_ref, n_ref, o_ref, nc_ref):
    nv = n_ref[...]
    sa = sa_ref[...].reshape(_BR, 1)
    sb = sb_ref[...].reshape(_BR, 1)
    o_ref[...] = jnp.clip(sa * x_ref[...] + sb * nv, -1.0, 1.0)
    nc_ref[...] = nv


def kernel(x_0, t, noise, sqrt_alpha, sqrt_1m_alpha):
    scale_a, scale_b = _make_sc_gather()(t, sqrt_alpha, sqrt_1m_alpha)
    out, n_copy = pl.pallas_call(
        _tc_body,
        grid=(B // _BR, D // _BC),
        in_specs=[
            pl.BlockSpec((_BR,), lambda i, j: (i,)),
            pl.BlockSpec((_BR,), lambda i, j: (i,)),
            pl.BlockSpec((_BR, _BC), lambda i, j: (i, j)),
            pl.BlockSpec((_BR, _BC), lambda i, j: (i, j)),
        ],
        out_specs=[
            pl.BlockSpec((_BR, _BC), lambda i, j: (i, j)),
            pl.BlockSpec((_BR, _BC), lambda i, j: (i, j)),
        ],
        out_shape=[
            jax.ShapeDtypeStruct((B, D), jnp.float32),
            jax.ShapeDtypeStruct((B, D), jnp.float32),
        ],
    )(scale_a, scale_b, x_0, noise)
    return out, n_copy


# speedup vs baseline: 1.0045x; 1.0007x over previous
import functools

import jax
import jax.numpy as jnp
from jax import lax
from jax.experimental import pallas as pl
from jax.experimental.pallas import tpu as pltpu
from jax.experimental.pallas import tpu_sc as plsc

B, D = 4096, 12288

_NC, _NS = 2, 16
_NW = _NC * _NS
_CHUNK = B // _NW
_LANES = 16


def _make_sc_gather():
    mesh = plsc.VectorSubcoreMesh(core_axis_name="c", subcore_axis_name="s")

    @functools.partial(
        pl.kernel,
        mesh=mesh,
        out_type=(
            jax.ShapeDtypeStruct((B,), jnp.float32),
            jax.ShapeDtypeStruct((B,), jnp.float32),
        ),
        scratch_types=[
            pltpu.VMEM((_CHUNK,), jnp.int32),
            pltpu.VMEM((_CHUNK,), jnp.float32),
            pltpu.VMEM((_CHUNK,), jnp.float32),
            pltpu.SemaphoreType.DMA,
            pltpu.SemaphoreType.DMA,
        ],
    )
    def sc_gather(t_hbm, sa_hbm, sb_hbm, oa_hbm, ob_hbm,
                  idx_v, oa_v, ob_v, sem_a, sem_b):
        wid = lax.axis_index("s") * _NC + lax.axis_index("c")
        base = wid * _CHUNK
        pltpu.sync_copy(t_hbm.at[pl.ds(base, _CHUNK)], idx_v)
        cp_a = pltpu.async_copy(sa_hbm.at[idx_v], oa_v, sem_a)
        cp_b = pltpu.async_copy(sb_hbm.at[idx_v], ob_v, sem_b)
        cp_a.wait()
        cp_b.wait()
        wb_a = pltpu.async_copy(oa_v, oa_hbm.at[pl.ds(base, _CHUNK)], sem_a)
        wb_b = pltpu.async_copy(ob_v, ob_hbm.at[pl.ds(base, _CHUNK)], sem_b)
        wb_a.wait()
        wb_b.wait()

    return sc_gather


_BR = 256
_BC = 4096


def _tc_body(sa_ref, sb_ref, x_ref, n_ref, o_ref, nc_ref):
    nv = n_ref[...]
    sa = sa_ref[...].reshape(_BR, 1)
    sb = sb_ref[...].reshape(_BR, 1)
    o_ref[...] = jnp.clip(sa * x_ref[...] + sb * nv, -1.0, 1.0)
    nc_ref[...] = nv


def kernel(x_0, t, noise, sqrt_alpha, sqrt_1m_alpha):
    scale_a, scale_b = _make_sc_gather()(t, sqrt_alpha, sqrt_1m_alpha)
    out, n_copy = pl.pallas_call(
        _tc_body,
        grid=(B // _BR, D // _BC),
        in_specs=[
            pl.BlockSpec((_BR,), lambda i, j: (i,)),
            pl.BlockSpec((_BR,), lambda i, j: (i,)),
            pl.BlockSpec((_BR, _BC), lambda i, j: (i, j)),
            pl.BlockSpec((_BR, _BC), lambda i, j: (i, j)),
        ],
        out_specs=[
            pl.BlockSpec((_BR, _BC), lambda i, j: (i, j)),
            pl.BlockSpec((_BR, _BC), lambda i, j: (i, j)),
        ],
        out_shape=[
            jax.ShapeDtypeStruct((B, D), jnp.float32),
            jax.ShapeDtypeStruct((B, D), jnp.float32),
        ],
    )(scale_a, scale_b, x_0, noise)
    return out, n_copy
